# Initial kernel scaffold; baseline (speedup 1.0000x reference)
import jax
import jax.numpy as jnp
from jax.experimental import pallas as pl


def kernel(z, codebook):
    raise NotImplementedError("write your pallas kernel here")


# speedup vs baseline: 1.2271x; 1.2271x over previous
import jax
import jax.numpy as jnp
from jax.experimental import pallas as pl
from jax.experimental.pallas import tpu as pltpu


def _vq_kernel(z_ref, cb_ref, cbt_ref, idx_ref, zq_ref):
    zb = z_ref[0]
    cb = cb_ref[...]
    zf = zb.T
    scores = jax.lax.dot_general(
        zf, cb, (((1,), (1,)), ((), ())),
        preferred_element_type=jnp.float32)
    zsq = jnp.sum(zf * zf, axis=1, keepdims=True)
    csq = jnp.sum(cb * cb, axis=1)
    dist = (zsq - 2.0 * scores) + csq[None, :]
    vmin = jnp.min(dist, axis=1, keepdims=True)
    kiota = jax.lax.broadcasted_iota(jnp.int32, dist.shape, 1)
    big = jnp.int32(dist.shape[1])
    idx = jnp.min(jnp.where(dist == vmin, kiota, big), axis=1)
    idx_ref[0, 0, :] = idx
    oh_iota = jax.lax.broadcasted_iota(jnp.int32, (cb.shape[0], idx.shape[0]), 0)
    onehot = (oh_iota == idx[None, :]).astype(jnp.float32)
    zq_ref[0] = jax.lax.dot_general(
        cbt_ref[...], onehot, (((1,), (0,)), ((), ())),
        preferred_element_type=jnp.float32)


def kernel(z, codebook):
    B, C, H, W = z.shape
    HW = H * W
    K = codebook.shape[0]
    zr = z.reshape(B, C, HW)
    cbt = codebook.T

    idx3, zq = pl.pallas_call(
        _vq_kernel,
        grid=(B,),
        in_specs=[
            pl.BlockSpec((1, C, HW), lambda b: (b, 0, 0)),
            pl.BlockSpec((K, C), lambda b: (0, 0)),
            pl.BlockSpec((C, K), lambda b: (0, 0)),
        ],
        out_specs=[
            pl.BlockSpec((1, 1, HW), lambda b: (b, 0, 0)),
            pl.BlockSpec((1, C, HW), lambda b: (b, 0, 0)),
        ],
        out_shape=[
            jax.ShapeDtypeStruct((B, 1, HW), jnp.int32),
            jax.ShapeDtypeStruct((B, C, HW), jnp.float32),
        ],
    )(zr, codebook, cbt)

    return zq.reshape(B, C, H, W), idx3.reshape(B, HW)


# speedup vs baseline: 1.2276x; 1.0004x over previous
import jax
import jax.numpy as jnp
from jax.experimental import pallas as pl
from jax.experimental.pallas import tpu as pltpu


def _vq_kernel(z_ref, cb_ref, cbt_ref, idx_ref, zq_ref):
    zb = z_ref[0]
    cb = cb_ref[...]
    zf = zb.T
    scores = jax.lax.dot_general(
        zf, cb, (((1,), (1,)), ((), ())),
        preferred_element_type=jnp.float32)
    zsq = jnp.sum(zf * zf, axis=1, keepdims=True)
    csq = jnp.sum(cb * cb, axis=1)
    dist = (zsq - 2.0 * scores) + csq[None, :]
    vmin = jnp.min(dist, axis=1, keepdims=True)
    kiota = jax.lax.broadcasted_iota(jnp.int32, dist.shape, 1)
    big = jnp.int32(dist.shape[1])
    idx = jnp.min(jnp.where(dist == vmin, kiota, big), axis=1)
    idx_ref[0, 0, :] = idx
    oh_iota = jax.lax.broadcasted_iota(jnp.int32, (cb.shape[0], idx.shape[0]), 0)
    onehot = (oh_iota == idx[None, :]).astype(jnp.bfloat16)
    zq_ref[0] = jax.lax.dot_general(
        cbt_ref[...], onehot, (((1,), (0,)), ((), ())),
        preferred_element_type=jnp.float32)


def kernel(z, codebook):
    B, C, H, W = z.shape
    HW = H * W
    K = codebook.shape[0]
    zr = z.reshape(B, C, HW)
    cbt = codebook.T.astype(jnp.bfloat16)

    idx3, zq = pl.pallas_call(
        _vq_kernel,
        grid=(B,),
        in_specs=[
            pl.BlockSpec((1, C, HW), lambda b: (b, 0, 0)),
            pl.BlockSpec((K, C), lambda b: (0, 0)),
            pl.BlockSpec((C, K), lambda b: (0, 0)),
        ],
        out_specs=[
            pl.BlockSpec((1, 1, HW), lambda b: (b, 0, 0)),
            pl.BlockSpec((1, C, HW), lambda b: (b, 0, 0)),
        ],
        out_shape=[
            jax.ShapeDtypeStruct((B, 1, HW), jnp.int32),
            jax.ShapeDtypeStruct((B, C, HW), jnp.float32),
        ],
    )(zr, codebook, cbt)

    return zq.reshape(B, C, H, W), idx3.reshape(B, HW)


# speedup vs baseline: 1.3287x; 1.0824x over previous
import jax
import jax.numpy as jnp
from jax.experimental import pallas as pl
from jax.experimental.pallas import tpu as pltpu


def _vq_kernel(z_ref, cb_ref, cbt_ref, idx_ref, zq_ref, csq_ref):
    cb = cb_ref[...]

    @pl.when(pl.program_id(0) == 0)
    def _init():
        csq_ref[...] = jnp.sum(cb * cb, axis=1, keepdims=True).T

    zb = z_ref[0]
    zf = zb.T
    zsq = jnp.sum(zb * zb, axis=0)[:, None]
    HW_ = zf.shape[0]
    K_ = cb.shape[0]
    CH = 256
    pval = None
    pj = None
    for j in range(K_ // CH):
        cbj = cb[j * CH:(j + 1) * CH, :]
        s = jax.lax.dot_general(
            zf, cbj, (((1,), (1,)), ((), ())),
            preferred_element_type=jnp.float32)
        dj = (zsq - 2.0 * s) + csq_ref[:, j * CH:(j + 1) * CH]
        if j == 0:
            pval = dj
            pj = jnp.zeros((HW_, CH), jnp.int32)
        else:
            lt = dj < pval
            pval = jnp.where(lt, dj, pval)
            pj = jnp.where(lt, jnp.int32(j), pj)
    lane = jax.lax.broadcasted_iota(jnp.int32, (HW_, CH), 1)
    pkey = pj * CH + lane
    tval = pval.T
    tkey = pkey.T
    m = jnp.min(tval, axis=0)[None, :]
    big = jnp.int32(K_)
    idx = jnp.min(jnp.where(tval == m, tkey, big), axis=0)
    idx_ref[0, 0, :] = idx
    oh_iota = jax.lax.broadcasted_iota(jnp.int32, (cb.shape[0], idx.shape[0]), 0)
    onehot = (oh_iota == idx[None, :]).astype(jnp.bfloat16)
    zq_ref[0] = jax.lax.dot_general(
        cbt_ref[...], onehot, (((1,), (0,)), ((), ())),
        preferred_element_type=jnp.float32)


def kernel(z, codebook):
    B, C, H, W = z.shape
    HW = H * W
    K = codebook.shape[0]
    zr = z.reshape(B, C, HW)
    cbt = codebook.T.astype(jnp.bfloat16)

    idx3, zq = pl.pallas_call(
        _vq_kernel,
        grid=(B,),
        in_specs=[
            pl.BlockSpec((1, C, HW), lambda b: (b, 0, 0)),
            pl.BlockSpec((K, C), lambda b: (0, 0)),
            pl.BlockSpec((C, K), lambda b: (0, 0)),
        ],
        out_specs=[
            pl.BlockSpec((1, 1, HW), lambda b: (b, 0, 0)),
            pl.BlockSpec((1, C, HW), lambda b: (b, 0, 0)),
        ],
        out_shape=[
            jax.ShapeDtypeStruct((B, 1, HW), jnp.int32),
            jax.ShapeDtypeStruct((B, C, HW), jnp.float32),
        ],
        scratch_shapes=[pltpu.VMEM((1, K), jnp.float32)],
    )(zr, codebook, cbt)

    return zq.reshape(B, C, H, W), idx3.reshape(B, HW)


# speedup vs baseline: 1.3687x; 1.0301x over previous
import jax
import jax.numpy as jnp
from jax.experimental import pallas as pl
from jax.experimental.pallas import tpu as pltpu

_BPB = 4


def _vq_body(zb, cb, cbt, csq_ref, idx_slot_ref, zq_slot_ref):
    zf = zb.T
    zsq = jnp.sum(zb * zb, axis=0)[:, None]
    HW_ = zf.shape[0]
    K_ = cb.shape[0]
    CH = 256
    pval = None
    pj = None
    for j in range(K_ // CH):
        cbj = cb[j * CH:(j + 1) * CH, :]
        s = jax.lax.dot_general(
            zf, cbj, (((1,), (1,)), ((), ())),
            preferred_element_type=jnp.float32)
        dj = (zsq - 2.0 * s) + csq_ref[:, j * CH:(j + 1) * CH]
        if j == 0:
            pval = dj
            pj = jnp.zeros((HW_, CH), jnp.int32)
        else:
            lt = dj < pval
            pval = jnp.where(lt, dj, pval)
            pj = jnp.where(lt, jnp.int32(j), pj)
    lane = jax.lax.broadcasted_iota(jnp.int32, (HW_, CH), 1)
    pkey = pj * CH + lane
    tval = pval.T
    tkey = pkey.T
    m = jnp.min(tval, axis=0)[None, :]
    big = jnp.int32(K_)
    idx = jnp.min(jnp.where(tval == m, tkey, big), axis=0)
    idx_slot_ref[...] = idx
    oh_iota = jax.lax.broadcasted_iota(jnp.int32, (K_, HW_), 0)
    onehot = (oh_iota == idx[None, :]).astype(jnp.bfloat16)
    zq_slot_ref[...] = jax.lax.dot_general(
        cbt, onehot, (((1,), (0,)), ((), ())),
        preferred_element_type=jnp.float32)


def _vq_kernel(z_ref, cb_ref, cbt_ref, idx_ref, zq_ref, csq_ref):
    cb = cb_ref[...]

    @pl.when(pl.program_id(0) == 0)
    def _init():
        csq_ref[...] = jnp.sum(cb * cb, axis=1, keepdims=True).T

    cbt = cbt_ref[...]
    for t in range(_BPB):
        _vq_body(z_ref[t], cb, cbt, csq_ref,
                 idx_ref.at[t, 0], zq_ref.at[t])


def kernel(z, codebook):
    B, C, H, W = z.shape
    HW = H * W
    K = codebook.shape[0]
    zr = z.reshape(B, C, HW)
    cbt = codebook.T.astype(jnp.bfloat16)
    nb = B // _BPB

    idx3, zq = pl.pallas_call(
        _vq_kernel,
        grid=(nb,),
        in_specs=[
            pl.BlockSpec((_BPB, C, HW), lambda b: (b, 0, 0)),
            pl.BlockSpec((K, C), lambda b: (0, 0)),
            pl.BlockSpec((C, K), lambda b: (0, 0)),
        ],
        out_specs=[
            pl.BlockSpec((_BPB, 1, HW), lambda b: (b, 0, 0)),
            pl.BlockSpec((_BPB, C, HW), lambda b: (b, 0, 0)),
        ],
        out_shape=[
            jax.ShapeDtypeStruct((B, 1, HW), jnp.int32),
            jax.ShapeDtypeStruct((B, C, HW), jnp.float32),
        ],
        scratch_shapes=[pltpu.VMEM((1, K), jnp.float32)],
    )(zr, codebook, cbt)

    return zq.reshape(B, C, H, W), idx3.reshape(B, HW)


# speedup vs baseline: 1.3976x; 1.0211x over previous
import functools
import jax
import jax.numpy as jnp
from jax import lax
from jax.experimental import pallas as pl
from jax.experimental.pallas import tpu as pltpu
from jax.experimental.pallas import tpu_sc as plsc

_BPB = 4


def _dist_body(zb, cb, csq_ref, idx_slot_ref):
    zf = zb.T
    zsq = jnp.sum(zb * zb, axis=0)[:, None]
    HW_ = zf.shape[0]
    K_ = cb.shape[0]
    CH = 256
    pval = None
    pj = None
    for j in range(K_ // CH):
        cbj = cb[j * CH:(j + 1) * CH, :]
        s = jax.lax.dot_general(
            zf, cbj, (((1,), (1,)), ((), ())),
            preferred_element_type=jnp.float32)
        dj = (zsq - 2.0 * s) + csq_ref[:, j * CH:(j + 1) * CH]
        if j == 0:
            pval = dj
            pj = jnp.zeros((HW_, CH), jnp.int32)
        else:
            lt = dj < pval
            pval = jnp.where(lt, dj, pval)
            pj = jnp.where(lt, jnp.int32(j), pj)
    lane = jax.lax.broadcasted_iota(jnp.int32, (HW_, CH), 1)
    pkey = pj * CH + lane
    tval = pval.T
    tkey = pkey.T
    m = jnp.min(tval, axis=0)[None, :]
    big = jnp.int32(K_)
    idx = jnp.min(jnp.where(tval == m, tkey, big), axis=0)
    idx_slot_ref[...] = idx


def _dist_kernel(z_ref, cb_ref, idx_ref, csq_ref):
    cb = cb_ref[...]

    @pl.when(pl.program_id(0) == 0)
    def _init():
        csq_ref[...] = jnp.sum(cb * cb, axis=1, keepdims=True).T

    for t in range(_BPB):
        _dist_body(z_ref[t], cb, csq_ref, idx_ref.at[t, 0])


def _make_sc_gather(K, C, N, NC, NS):
    NW = NC * NS
    n_per_w = N // NW
    CHG = 128
    n_chunks = n_per_w // CHG
    mesh = plsc.VectorSubcoreMesh(core_axis_name="c", subcore_axis_name="s")

    @functools.partial(
        pl.kernel, mesh=mesh,
        out_type=jax.ShapeDtypeStruct((N, C), jnp.float32),
        scratch_types=[
            pltpu.VMEM((CHG,), jnp.int32),
            pltpu.VMEM((CHG, C), jnp.float32),
            pltpu.SemaphoreType.DMA,
        ],
    )
    def gather_k(cb_hbm, idx_hbm, out_hbm, idx_v, rows_v, sem):
        wid = lax.axis_index("s") * NC + lax.axis_index("c")
        base = wid * n_per_w
        for chunk in range(n_chunks):
            off = base + chunk * CHG
            pltpu.sync_copy(idx_hbm.at[pl.ds(off, CHG)], idx_v)
            pltpu.async_copy(cb_hbm.at[idx_v], rows_v, sem).wait()
            pltpu.sync_copy(rows_v, out_hbm.at[pl.ds(off, CHG)])

    return gather_k


def kernel(z, codebook):
    B, C, H, W = z.shape
    HW = H * W
    K = codebook.shape[0]
    N = B * HW
    zr = z.reshape(B, C, HW)
    nb = B // _BPB

    idx3 = pl.pallas_call(
        _dist_kernel,
        grid=(nb,),
        in_specs=[
            pl.BlockSpec((_BPB, C, HW), lambda b: (b, 0, 0)),
            pl.BlockSpec((K, C), lambda b: (0, 0)),
        ],
        out_specs=pl.BlockSpec((_BPB, 1, HW), lambda b: (b, 0, 0)),
        out_shape=jax.ShapeDtypeStruct((B, 1, HW), jnp.int32),
        scratch_shapes=[pltpu.VMEM((1, K), jnp.float32)],
    )(zr, codebook)

    info = plsc.get_sparse_core_info()
    gather_k = _make_sc_gather(K, C, N, info.num_cores, info.num_subcores)
    zq_flat = gather_k(codebook, idx3.reshape(N))

    zq = zq_flat.reshape(B, HW, C).transpose(0, 2, 1).reshape(B, C, H, W)
    return zq, idx3.reshape(B, HW)
